# baseline (speedup 1.0000x reference)
import functools

import jax
import jax.numpy as jnp
from jax import lax
from jax.experimental import pallas as pl
from jax.experimental.pallas import tpu as pltpu
from jax.experimental.pallas import tpu_sc as plsc

N = 10000
E = 160000
D = 256
H = D // 2
NSUB = 16
EPT = E // NSUB
K = 128
NCH = 80
EPTP = NCH * K
ZR = 80
P0 = 5120
ACC = 5200
DUMMY = P0
PHASES = ((0, P0), (P0, N - P0))


def _sc_message_passing(xflat, src, dstg, wg):
    mesh = plsc.VectorSubcoreMesh(core_axis_name="c", subcore_axis_name="s")

    def bcast_lane(vec16, lane):
        idx = jnp.full((16,), lane, jnp.int32)
        return lax.gather(
            vec16, idx[:, None],
            lax.GatherDimensionNumbers(offset_dims=(), collapsed_slice_dims=(0,),
                                       start_index_map=(0,)),
            (1,), mode=lax.GatherScatterMode.PROMISE_IN_BOUNDS)

    @functools.partial(
        pl.kernel,
        out_type=jax.ShapeDtypeStruct((2, N, H), jnp.float32),
        mesh=mesh,
        scratch_types=[
            pltpu.VMEM((EPTP,), jnp.int32),
            pltpu.VMEM((NCH, K), jnp.int32),
            pltpu.VMEM((NCH, K), jnp.int32),
            pltpu.VMEM((EPTP,), jnp.float32),
            pltpu.VMEM((K, H), jnp.float32),
            pltpu.VMEM((K, H), jnp.float32),
            pltpu.VMEM((ZR, H), jnp.float32),
            pltpu.VMEM_SHARED((ACC, H), jnp.float32),
            pltpu.SemaphoreType.DMA,
            pltpu.SemaphoreType.DMA,
        ],
    )
    def sc_kernel(xflat_hbm, src_hbm, dstg_hbm, wg_hbm, out_hbm,
                  src_v, dstr_v, dst_v, w_v, rows_a, rows_b, zero_v, acc_sh,
                  sem_a, sem_b):
        c = lax.axis_index("c")
        s = lax.axis_index("s")

        pltpu.sync_copy(src_hbm.at[s], src_v)
        pltpu.sync_copy(dstg_hbm.at[s], dstr_v)
        pltpu.sync_copy(wg_hbm.at[s], w_v)

        def sfix(i, _):
            sl = pl.ds(pl.multiple_of(i * 16, 16), 16)
            src_v[sl] = src_v[sl] * 2 + c
            return _
        lax.fori_loop(0, EPTP // 16, sfix, None)

        def zfill(r, _):
            for cc in range(H // 16):
                zero_v[r, pl.ds(cc * 16, 16)] = jnp.zeros((16,), jnp.float32)
            return _
        lax.fori_loop(0, ZR, zfill, None)

        for base, size in PHASES:
            def dfix(j, _):
                for g in range(K // 16):
                    sl = pl.ds(g * 16, 16)
                    d = dstr_v[j, sl]
                    ok = jnp.logical_and(d >= base, d < base + size)
                    dst_v[j, sl] = jnp.where(ok, d - base,
                                             jnp.full((16,), DUMMY, jnp.int32))
                return _
            lax.fori_loop(0, NCH, dfix, None)

            nz = ACC // ZR
            def zcopy(i, _):
                t = s + i * NSUB
                pltpu.sync_copy(zero_v, acc_sh.at[pl.ds(t * ZR, ZR)])
                return _
            lax.fori_loop(0, (nz - s + NSUB - 1) // NSUB, zcopy, None)
            plsc.subcore_barrier()

            def gstart(j, buf, sem):
                jb = pl.ds(pl.multiple_of(j * K, K), K)
                pltpu.async_copy(xflat_hbm.at[src_v.at[jb]], buf, sem)

            def gwait(buf, sem):
                pltpu.make_async_copy(xflat_hbm.at[pl.ds(0, K)], buf, sem).wait()

            def scale(buf, j):
                def group(g, __):
                    wb16 = w_v[pl.ds(pl.multiple_of(j * K + g * 16, 16), 16)]
                    for e in range(16):
                        wb = bcast_lane(wb16, e)
                        r = g * 16 + e
                        for cc in range(H // 16):
                            sl = pl.ds(cc * 16, 16)
                            buf[r, sl] = buf[r, sl] * wb
                    return __
                lax.fori_loop(0, K // 16, group, None)

            gstart(0, rows_a, sem_a)

            def pair(j2, _):
                j = j2 * 2
                gwait(rows_a, sem_a)
                gstart(j + 1, rows_b, sem_b)
                scale(rows_a, j)
                pltpu.sync_copy(rows_a, acc_sh.at[dst_v.at[j]], add=True)

                @pl.when(j + 2 < NCH)
                def _prefetch():
                    gstart(j + 2, rows_a, sem_a)
                gwait(rows_b, sem_b)
                scale(rows_b, j + 1)
                pltpu.sync_copy(rows_b, acc_sh.at[dst_v.at[j + 1]], add=True)
                return _
            lax.fori_loop(0, NCH // 2, pair, None)
            plsc.subcore_barrier()

            nw = size // ZR
            def wb(i, _):
                t = s + i * NSUB
                pltpu.sync_copy(acc_sh.at[pl.ds(t * ZR, ZR)],
                                out_hbm.at[c, pl.ds(base + t * ZR, ZR)])
                return _
            lax.fori_loop(0, (nw - s + NSUB - 1) // NSUB, wb, None)

    return sc_kernel(xflat, src, dstg, wg)


def _tc_conv(x, agg2, w_self, w_nbr, b, w_post=None, b_post=None):
    R = 1000
    grid = (N // R,)
    final = w_post is not None

    def body(x_ref, a0_ref, a1_ref, ws_ref, wn_ref, b_ref, *rest):
        if final:
            wp_ref, bp_ref, o_ref = rest
        else:
            (o_ref,) = rest
        h = jnp.dot(x_ref[...], ws_ref[...], preferred_element_type=jnp.float32)
        h += jnp.dot(a0_ref[0], wn_ref[:H, :], preferred_element_type=jnp.float32)
        h += jnp.dot(a1_ref[0], wn_ref[H:, :], preferred_element_type=jnp.float32)
        h += b_ref[...]
        h = jnp.maximum(h, 0.0)
        if final:
            h = jnp.dot(h, wp_ref[...], preferred_element_type=jnp.float32)
            h += bp_ref[...]
        o_ref[...] = h

    in_specs = [
        pl.BlockSpec((R, D), lambda i: (i, 0)),
        pl.BlockSpec((1, R, H), lambda i: (0, i, 0)),
        pl.BlockSpec((1, R, H), lambda i: (1, i, 0)),
        pl.BlockSpec((D, D), lambda i: (0, 0)),
        pl.BlockSpec((D, D), lambda i: (0, 0)),
        pl.BlockSpec((1, D), lambda i: (0, 0)),
    ]
    args = [x, agg2, agg2, w_self, w_nbr, b.reshape(1, D)]
    if final:
        in_specs += [pl.BlockSpec((D, D), lambda i: (0, 0)),
                     pl.BlockSpec((1, D), lambda i: (0, 0))]
        args += [w_post, b_post.reshape(1, D)]

    return pl.pallas_call(
        body,
        grid=grid,
        in_specs=in_specs,
        out_specs=pl.BlockSpec((R, D), lambda i: (i, 0)),
        out_shape=jax.ShapeDtypeStruct((N, D), jnp.float32),
    )(*args)


def kernel(h5_all, edge_weight, W6_self, W6_nbr, b6, W7_self, W7_nbr, b7,
           W_post, b_post, edge_index):
    pad = ((0, 0), (0, EPTP - EPT))
    src = jnp.pad(edge_index[0].astype(jnp.int32).reshape(NSUB, EPT), pad)
    dstg = jnp.pad(edge_index[1].astype(jnp.int32).reshape(NSUB, EPT),
                   pad).reshape(NSUB, NCH, K)
    wg = jnp.pad(edge_weight.reshape(NSUB, EPT), pad)

    agg6 = _sc_message_passing(h5_all.reshape(2 * N, H), src, dstg, wg)
    h6 = _tc_conv(h5_all, agg6, W6_self, W6_nbr, b6)
    agg7 = _sc_message_passing(h6.reshape(2 * N, H), src, dstg, wg)
    out = _tc_conv(h6, agg7, W7_self, W7_nbr, b7, W_post, b_post)
    return out


# speedup vs baseline: 1.8683x; 1.8683x over previous
import functools

import jax
import jax.numpy as jnp
from jax import lax
from jax.experimental import pallas as pl
from jax.experimental.pallas import tpu as pltpu
from jax.experimental.pallas import tpu_sc as plsc

N = 10000
E = 160000
D = 256
H = D // 2
NSUB = 16
EPT = E // NSUB
K = 80
NCH = EPT // K
ZR = 80
P0 = 5120
ACC = 5200
DUMMY = P0
PHASES = ((0, P0), (P0, N - P0))


def _sc_message_passing(xflat, src, dstg, wg):
    mesh = plsc.VectorSubcoreMesh(core_axis_name="c", subcore_axis_name="s")

    def bcast_lane(vec16, lane):
        idx = jnp.full((16,), lane, jnp.int32)
        return lax.gather(
            vec16, idx[:, None],
            lax.GatherDimensionNumbers(offset_dims=(), collapsed_slice_dims=(0,),
                                       start_index_map=(0,)),
            (1,), mode=lax.GatherScatterMode.PROMISE_IN_BOUNDS)

    @functools.partial(
        pl.kernel,
        out_type=jax.ShapeDtypeStruct((2, N, H), jnp.float32),
        mesh=mesh,
        scratch_types=[
            pltpu.VMEM((EPT,), jnp.int32),
            pltpu.VMEM((NCH, K), jnp.int32),
            pltpu.VMEM((NCH, K), jnp.int32),
            pltpu.VMEM((EPT,), jnp.float32),
            pltpu.VMEM((K, H), jnp.float32),
            pltpu.VMEM((K, H), jnp.float32),
            pltpu.VMEM((ZR, H), jnp.float32),
            pltpu.VMEM_SHARED((ACC, H), jnp.float32),
            pltpu.SemaphoreType.DMA,
            pltpu.SemaphoreType.DMA,
        ],
    )
    def sc_kernel(xflat_hbm, src_hbm, dstg_hbm, wg_hbm, out_hbm,
                  src_v, dstr_v, dst_v, w_v, rows_a, rows_b, zero_v, acc_sh, sem_a, sem_b):
        c = lax.axis_index("c")
        s = lax.axis_index("s")

        pltpu.sync_copy(src_hbm.at[s], src_v)
        pltpu.sync_copy(dstg_hbm.at[s], dstr_v)
        pltpu.sync_copy(wg_hbm.at[s], w_v)

        def sfix(i, _):
            sl = pl.ds(pl.multiple_of(i * 16, 16), 16)
            src_v[sl] = src_v[sl] * 2 + c
            return _
        lax.fori_loop(0, EPT // 16, sfix, None)

        def zfill(r, _):
            for cc in range(H // 16):
                zero_v[r, pl.ds(cc * 16, 16)] = jnp.zeros((16,), jnp.float32)
            return _
        lax.fori_loop(0, ZR, zfill, None)

        for base, size in PHASES:
            def dfix(j, _):
                for g in range(K // 16):
                    sl = pl.ds(g * 16, 16)
                    d = dstr_v[j, sl]
                    ok = jnp.logical_and(d >= base, d < base + size)
                    dst_v[j, sl] = jnp.where(ok, d - base,
                                             jnp.full((16,), DUMMY, jnp.int32))
                return _
            lax.fori_loop(0, NCH, dfix, None)

            nz = ACC // ZR
            def zcopy(i, _):
                t = s + i * NSUB
                pltpu.sync_copy(zero_v, acc_sh.at[pl.ds(t * ZR, ZR)])
                return _
            lax.fori_loop(0, (nz - s + NSUB - 1) // NSUB, zcopy, None)
            plsc.subcore_barrier()

            def gstart(j, buf, sem):
                jb = pl.ds(pl.multiple_of(j * K, K), K)
                pltpu.async_copy(xflat_hbm.at[src_v.at[jb]], buf, sem)

            def gwait(buf, sem):
                pltpu.make_async_copy(xflat_hbm.at[pl.ds(0, K)], buf, sem).wait()

            def scale(buf, j):
                def group(g, __):
                    wb16 = w_v[pl.ds(pl.multiple_of(j * K + g * 16, 16), 16)]
                    for e in range(16):
                        wb = bcast_lane(wb16, e)
                        r = g * 16 + e
                        for cc in range(H // 16):
                            sl = pl.ds(cc * 16, 16)
                            buf[r, sl] = buf[r, sl] * wb
                    return __
                lax.fori_loop(0, K // 16, group, None)

            gstart(0, rows_a, sem_a)

            def pair(j2, _):
                j = j2 * 2
                gwait(rows_a, sem_a)
                gstart(j + 1, rows_b, sem_b)
                scale(rows_a, j)
                pltpu.sync_copy(rows_a, acc_sh.at[dst_v.at[j]], add=True)
                gstart(j + 2, rows_a, sem_a)
                gwait(rows_b, sem_b)
                scale(rows_b, j + 1)
                pltpu.sync_copy(rows_b, acc_sh.at[dst_v.at[j + 1]], add=True)
                return _
            lax.fori_loop(0, NCH // 2, pair, None)
            gwait(rows_a, sem_a)
            scale(rows_a, NCH - 1)
            pltpu.sync_copy(rows_a, acc_sh.at[dst_v.at[NCH - 1]], add=True)
            plsc.subcore_barrier()

            nw = size // ZR
            def wb(i, _):
                t = s + i * NSUB
                pltpu.sync_copy(acc_sh.at[pl.ds(t * ZR, ZR)],
                                out_hbm.at[c, pl.ds(base + t * ZR, ZR)])
                return _
            lax.fori_loop(0, (nw - s + NSUB - 1) // NSUB, wb, None)

    return sc_kernel(xflat, src, dstg, wg)


def _tc_conv(x, agg2, w_self, w_nbr, b, w_post=None, b_post=None):
    R = 1000
    grid = (N // R,)
    final = w_post is not None

    def body(x_ref, a0_ref, a1_ref, ws_ref, wn_ref, b_ref, *rest):
        if final:
            wp_ref, bp_ref, o_ref = rest
        else:
            (o_ref,) = rest
        h = jnp.dot(x_ref[...], ws_ref[...], preferred_element_type=jnp.float32)
        h += jnp.dot(a0_ref[0], wn_ref[:H, :], preferred_element_type=jnp.float32)
        h += jnp.dot(a1_ref[0], wn_ref[H:, :], preferred_element_type=jnp.float32)
        h += b_ref[...]
        h = jnp.maximum(h, 0.0)
        if final:
            h = jnp.dot(h, wp_ref[...], preferred_element_type=jnp.float32)
            h += bp_ref[...]
        o_ref[...] = h

    in_specs = [
        pl.BlockSpec((R, D), lambda i: (i, 0)),
        pl.BlockSpec((1, R, H), lambda i: (0, i, 0)),
        pl.BlockSpec((1, R, H), lambda i: (1, i, 0)),
        pl.BlockSpec((D, D), lambda i: (0, 0)),
        pl.BlockSpec((D, D), lambda i: (0, 0)),
        pl.BlockSpec((1, D), lambda i: (0, 0)),
    ]
    args = [x, agg2, agg2, w_self, w_nbr, b.reshape(1, D)]
    if final:
        in_specs += [pl.BlockSpec((D, D), lambda i: (0, 0)),
                     pl.BlockSpec((1, D), lambda i: (0, 0))]
        args += [w_post, b_post.reshape(1, D)]

    return pl.pallas_call(
        body,
        grid=grid,
        in_specs=in_specs,
        out_specs=pl.BlockSpec((R, D), lambda i: (i, 0)),
        out_shape=jax.ShapeDtypeStruct((N, D), jnp.float32),
    )(*args)


def kernel(h5_all, edge_weight, W6_self, W6_nbr, b6, W7_self, W7_nbr, b7,
           W_post, b_post, edge_index):
    src = edge_index[0].astype(jnp.int32).reshape(NSUB, EPT)
    dstg = edge_index[1].astype(jnp.int32).reshape(NSUB, NCH, K)
    wg = edge_weight.reshape(NSUB, EPT)

    agg6 = _sc_message_passing(h5_all.reshape(2 * N, H), src, dstg, wg)
    h6 = _tc_conv(h5_all, agg6, W6_self, W6_nbr, b6)
    agg7 = _sc_message_passing(h6.reshape(2 * N, H), src, dstg, wg)
    out = _tc_conv(h6, agg7, W7_self, W7_nbr, b7, W_post, b_post)
    return out
